# baseline (speedup 1.0000x reference)
import functools

import jax
import jax.numpy as jnp
from jax import lax
from jax.experimental import pallas as pl
from jax.experimental.pallas import tpu as pltpu
from jax.experimental.pallas import tpu_sc as plsc

CUTOFF = 5.0
_EA, _EB, _EC = -28.0, 48.0, -21.0

_NC, _NS, _L = 2, 16, 16
_NW = _NC * _NS


def _pick(n):
    for b in (2000, 1600, 1000, 800, 400, 200, 100, 80, 64, 40, 32, 16, 8, 4, 2, 1):
        if n % b == 0:
            return b
    return 1


def _swish(x):
    return x * jax.nn.sigmoid(x)


def _envelope(x):
    x2 = x * x
    x4 = x2 * x2
    x5 = x4 * x
    return 1.0 / x + x5 * (_EA + x * (_EB + x * _EC))


def _emb_body(z_ref, emb_ref, x_ref):
    z = z_ref[...]
    zcap = emb_ref.shape[0]
    ids = lax.broadcasted_iota(jnp.int32, (1, zcap), 1)
    onehot = (z == ids).astype(jnp.float32)
    x_ref[...] = jnp.dot(onehot, emb_ref[...], preferred_element_type=jnp.float32)


def _node_embed(z2, embpad, bn):
    n = z2.shape[0]
    zcap, emb = embpad.shape
    return pl.pallas_call(
        _emb_body,
        grid=(n // bn,),
        in_specs=[
            pl.BlockSpec((bn, 1), lambda i: (i, 0)),
            pl.BlockSpec((zcap, emb), lambda i: (0, 0)),
        ],
        out_specs=pl.BlockSpec((bn, emb), lambda i: (i, 0)),
        out_shape=jax.ShapeDtypeStruct((n, emb), jnp.float32),
    )(z2, embpad)


def _edge_body(d_ref, xs_ref, xd_ref, freq_ref, wre_ref, w1_ref, w2_ref, w3_ref,
               wji_ref, wkj_ref, wr1_ref, wro_ref, wrp_ref,
               m_ref, mji_ref, mkx_ref, rbo_ref):
    ds = d_ref[...] * (1.0 / CUTOFF)
    env = _envelope(ds)
    rbf = env * jnp.sin(ds * freq_ref[...])
    rbf_e = _swish(jnp.dot(rbf, wre_ref[...], preferred_element_type=jnp.float32))
    pre = (jnp.dot(xs_ref[...], w1_ref[...], preferred_element_type=jnp.float32)
           + jnp.dot(xd_ref[...], w2_ref[...], preferred_element_type=jnp.float32)
           + jnp.dot(rbf_e, w3_ref[...], preferred_element_type=jnp.float32))
    m = _swish(pre)
    m_ref[...] = m
    mji_ref[...] = _swish(jnp.dot(m, wji_ref[...], preferred_element_type=jnp.float32))
    mkj = _swish(jnp.dot(m, wkj_ref[...], preferred_element_type=jnp.float32)) * \
        jnp.dot(rbf, wr1_ref[...], preferred_element_type=jnp.float32)
    a6 = jnp.dot(rbf, wrp_ref[...], preferred_element_type=jnp.float32)
    mkx_ref[...] = jnp.concatenate([mkj, a6], axis=1)
    rbo_ref[...] = jnp.dot(rbf, wro_ref[...], preferred_element_type=jnp.float32)


def _edge_block(d2, xs, xd, freq2, W_rbf_emb, W1, W2, W3, W_ji, W_kj, W_rbf1,
                W_rbf_out, Wrp, be):
    e = d2.shape[0]
    emb = xs.shape[1]
    full = lambda a: pl.BlockSpec(a.shape, lambda i: (0,) * a.ndim)
    row = lambda w: pl.BlockSpec((be, w), lambda i: (i, 0))
    return pl.pallas_call(
        _edge_body,
        grid=(e // be,),
        in_specs=[row(1), row(emb), row(emb), full(freq2), full(W_rbf_emb),
                  full(W1), full(W2), full(W3), full(W_ji), full(W_kj),
                  full(W_rbf1), full(W_rbf_out), full(Wrp)],
        out_specs=[row(emb), row(emb), row(2 * emb), row(emb)],
        out_shape=[
            jax.ShapeDtypeStruct((e, emb), jnp.float32),
            jax.ShapeDtypeStruct((e, emb), jnp.float32),
            jax.ShapeDtypeStruct((e, 2 * emb), jnp.float32),
            jax.ShapeDtypeStruct((e, emb), jnp.float32),
        ],
    )(d2, xs, xd, freq2, W_rbf_emb, W1, W2, W3, W_ji, W_kj, W_rbf1, W_rbf_out,
      Wrp)


def _trip_body(xkx_ref, ang_ref, s_ref, wb_ref, t_ref):
    emb = t_ref.shape[1]
    xe = xkx_ref[...]
    xk = xe[:, :emb]
    cbf = jnp.cos(ang_ref[...] * s_ref[...])
    sbf_p = cbf[:, 0:1] * xe[:, emb:emb + 8]
    for s in range(1, 7):
        sbf_p = sbf_p + cbf[:, s:s + 1] * xe[:, emb + 8 * s:emb + 8 * s + 8]
    y = jnp.dot(xk, wb_ref[...], preferred_element_type=jnp.float32)
    t = sbf_p[:, 0:1] * y[:, :emb]
    for l in range(1, 8):
        t = t + sbf_p[:, l:l + 1] * y[:, emb * l:emb * (l + 1)]
    t_ref[...] = t


def _triplet_block(xkx, ang2, s2, Wbil2, bt):
    tt = ang2.shape[0]
    emb = Wbil2.shape[0]
    full = lambda a: pl.BlockSpec(a.shape, lambda i: (0,) * a.ndim)
    row = lambda w: pl.BlockSpec((bt, w), lambda i: (i, 0))
    return pl.pallas_call(
        _trip_body,
        grid=(tt // bt,),
        in_specs=[row(2 * emb), row(1), full(s2), full(Wbil2)],
        out_specs=row(emb),
        out_shape=jax.ShapeDtypeStruct((tt, emb), jnp.float32),
    )(xkx, ang2, s2, Wbil2)


def _post_body(m_ref, mji_ref, agg_ref, rbo_ref, wa_ref, wb_ref, wc_ref, wd_ref,
               g_ref):
    m2 = mji_ref[...] + agg_ref[...]
    h = m2 + _swish(jnp.dot(
        _swish(jnp.dot(m2, wa_ref[...], preferred_element_type=jnp.float32)),
        wb_ref[...], preferred_element_type=jnp.float32))
    m3 = h + m_ref[...]
    m4 = m3 + _swish(jnp.dot(
        _swish(jnp.dot(m3, wc_ref[...], preferred_element_type=jnp.float32)),
        wd_ref[...], preferred_element_type=jnp.float32))
    g_ref[...] = rbo_ref[...] * m4


def _post_block(m, mji, agg, rbo, Wa, Wb, Wc, Wd, be):
    e, emb = m.shape
    full = lambda a: pl.BlockSpec(a.shape, lambda i: (0,) * a.ndim)
    row = lambda w: pl.BlockSpec((be, w), lambda i: (i, 0))
    return pl.pallas_call(
        _post_body,
        grid=(e // be,),
        in_specs=[row(emb), row(emb), row(emb), row(emb), full(Wa), full(Wb),
                  full(Wc), full(Wd)],
        out_specs=row(emb),
        out_shape=jax.ShapeDtypeStruct((e, emb), jnp.float32),
    )(m, mji, agg, rbo, Wa, Wb, Wc, Wd)


def _out_body(node_ref, w1_ref, w2_ref, o_ref):
    n2 = _swish(jnp.dot(node_ref[...], w1_ref[...],
                        preferred_element_type=jnp.float32))
    o_ref[...] = jnp.dot(n2, w2_ref[...], preferred_element_type=jnp.float32)


def _out_block(node, W1, W2, bn):
    n, emb = node.shape
    full = lambda a: pl.BlockSpec(a.shape, lambda i: (0,) * a.ndim)
    return pl.pallas_call(
        _out_body,
        grid=(n // bn,),
        in_specs=[pl.BlockSpec((bn, emb), lambda i: (i, 0)), full(W1), full(W2)],
        out_specs=pl.BlockSpec((bn, 1), lambda i: (i, 0)),
        out_shape=jax.ShapeDtypeStruct((n, 1), jnp.float32),
    )(node, W1, W2)


_GDN = lax.GatherDimensionNumbers(offset_dims=(), collapsed_slice_dims=(0,),
                                  start_index_map=(0,))


def _shuf(x, idx):
    return lax.gather(x, idx[:, None], _GDN, slice_sizes=(1,),
                      mode=lax.GatherScatterMode.PROMISE_IN_BOUNDS)


def _lanesum(x, iota16):
    for kk in (8, 4, 2, 1):
        x = x + _shuf(x, iota16 ^ kk)
    return x[0]


def _prefix(x, iota16):
    zero = jnp.zeros((16,), jnp.int32)
    for kk in (1, 2, 4, 8):
        kkv = jnp.full((16,), kk, jnp.int32)
        sh = _shuf(x, jnp.maximum(iota16 - kkv, zero))
        x = x + jnp.where(iota16 >= kkv, sh, zero)
    return x


def _gather_rows(table, idx):
    v, dd = table.shape
    b = idx.shape[0]
    assert b % _NW == 0 and dd % 16 == 0
    per_w = b // _NW
    chunk = per_w
    for c in (512, 400, 256, 200, 128, 80, 64, 40, 16, 8):
        if per_w % c == 0 and c * dd * 4 <= 280 * 1024:
            chunk = c
            break
    iters = per_w // chunk
    mesh = plsc.VectorSubcoreMesh(core_axis_name="c", subcore_axis_name="s")

    @functools.partial(
        pl.kernel, mesh=mesh,
        out_type=jax.ShapeDtypeStruct((b, dd), jnp.float32),
        scratch_types=[
            pltpu.VMEM((chunk,), jnp.int32),
            pltpu.VMEM((chunk, dd), jnp.float32),
            pltpu.SemaphoreType.DMA,
        ],
    )
    def k(table_hbm, idx_hbm, out_hbm, idx_v, rows_v, sem):
        wid = lax.axis_index("s") * _NC + lax.axis_index("c")
        base = wid * per_w

        def body(i, carry):
            off = base + i * chunk
            pltpu.sync_copy(idx_hbm.at[pl.ds(off, chunk)], idx_v)
            pltpu.async_copy(table_hbm.at[idx_v], rows_v, sem).wait()
            pltpu.sync_copy(rows_v, out_hbm.at[pl.ds(off, chunk)])
            return carry

        lax.fori_loop(0, iters, body, 0)

    return k(table, idx)


def _scatter_add_rows(rows, idx, nout):
    tin, dd = rows.shape
    assert tin % (_NS * 16) == 0
    per_tile = tin // _NS
    cap = per_tile + 128
    cmax = (5 * 2**19) // (dd * 4) // 128 * 128
    chalf = -(-nout // 2)
    csz = min(cmax, -(-chalf // 128) * 128)
    passes = -(-nout // (2 * csz))
    npad = 2 * csz * passes
    sch = 2000
    while per_tile % sch or sch % 16:
        sch //= 2
    nchunks = per_tile // sch
    spl = csz // 16
    zrows = 128
    mesh = plsc.VectorSubcoreMesh(core_axis_name="c", subcore_axis_name="s")
    zeros = jnp.zeros((zrows, dd), jnp.float32)

    @functools.partial(
        pl.kernel, mesh=mesh,
        out_type=jax.ShapeDtypeStruct((npad, dd), jnp.float32),
        scratch_types=[
            pltpu.VMEM((sch,), jnp.int32),
            pltpu.VMEM((cap,), jnp.int32),
            pltpu.VMEM((cap,), jnp.int32),
            pltpu.VMEM((1, 128), jnp.int32),
            pltpu.VMEM((128, dd), jnp.float32),
            pltpu.VMEM((zrows, dd), jnp.float32),
            pltpu.VMEM_SHARED((csz + 128, dd), jnp.float32),
            pltpu.SemaphoreType.DMA,
        ],
    )
    def k(rows_hbm, idx_hbm, zeros_hbm, out_hbm,
          idx_v, wbuf, tbuf, tstage, rows_v, zero_v, shared, sem):
        cid = lax.axis_index("c")
        sid = lax.axis_index("s")
        scan_base = sid * per_tile
        pltpu.sync_copy(zeros_hbm, zero_v)
        iota16 = lax.broadcasted_iota(jnp.int32, (16,), 0)

        for p in range(passes):
            lo = (p * _NC + cid) * csz
            zfull, zrem = spl // zrows, spl % zrows
            for q in range(zfull):
                pltpu.sync_copy(zero_v,
                                shared.at[pl.ds(sid * spl + q * zrows, zrows)])
            if zrem:
                pltpu.sync_copy(zero_v.at[pl.ds(0, zrem)],
                                shared.at[pl.ds(sid * spl + zfull * zrows, zrem)])
            plsc.subcore_barrier()

            def chunk_body(c, n):
                pltpu.sync_copy(idx_hbm.at[pl.ds(scan_base + c * sch, sch)],
                                idx_v)

                def vec_body(vv, n):
                    vec = idx_v[pl.ds(vv * 16, 16)]
                    lov = jnp.full((16,), lo, jnp.int32)
                    hiv = jnp.full((16,), lo + csz, jnp.int32)
                    msk = (vec >= lov) & (vec < hiv)
                    mi = jnp.where(msk, jnp.full((16,), 1, jnp.int32),
                                   jnp.full((16,), 0, jnp.int32))
                    cum = _prefix(mi, iota16)
                    jp1 = iota16 + jnp.full((16,), 1, jnp.int32)
                    inv = jnp.zeros((16,), jnp.int32)
                    for ss in (8, 4, 2, 1):
                        tt = inv + jnp.full((16,), ss, jnp.int32)
                        ci = _shuf(cum, tt - jnp.full((16,), 1, jnp.int32))
                        inv = jnp.where(ci < jp1, tt, inv)
                    wglob = jnp.full((16,), scan_base + c * sch + vv * 16,
                                     jnp.int32) + iota16
                    tbuf[pl.ds(n, 16)] = _shuf(vec - lov, inv)
                    wbuf[pl.ds(n, 16)] = _shuf(wglob, inv)
                    return n + cum[15]

                return lax.fori_loop(0, sch // 16, vec_body, n)

            n = lax.fori_loop(0, nchunks, chunk_body, jnp.int32(0))
            for kk in range(8):
                tbuf[pl.ds(n + kk * 16, 16)] = jnp.full((16,), csz, jnp.int32)
                wbuf[pl.ds(n + kk * 16, 16)] = jnp.zeros((16,), jnp.int32)

            def flush_body(g, carry):
                for kk in range(8):
                    tstage[0, pl.ds(kk * 16, 16)] = \
                        tbuf[pl.ds(g * 128 + kk * 16, 16)]
                pltpu.async_copy(rows_hbm.at[wbuf.at[pl.ds(g * 128, 128)]],
                                 rows_v, sem).wait()
                pltpu.sync_copy(rows_v, shared.at[tstage.at[0]], add=True)
                return carry

            lax.fori_loop(0, (n + 127) // 128, flush_body, 0)
            plsc.subcore_barrier()

            pltpu.sync_copy(shared.at[pl.ds(sid * spl, spl)],
                            out_hbm.at[pl.ds(lo + sid * spl, spl)])

    return k(rows, idx, zeros)[:nout]


def kernel(z, edge_index, d, angle, idx_kj, idx_ji, emb_table, freq, W_rbf_emb,
           W_cat, W_rbf1, W_sbf1, W_kj, W_ji, W_bil, W_res1a, W_res1b, W_res2a,
           W_res2b, W_rbf_out, W_out1, W_out2):
    n = z.shape[0]
    e = d.shape[0]
    t = angle.shape[0]
    emb = emb_table.shape[1]
    r = freq.shape[0]
    nb = W_bil.shape[2]
    s = W_sbf1.shape[0] // r

    src = edge_index[0]
    dst = edge_index[1]

    zcap = ((emb_table.shape[0] + 7) // 8) * 8
    embpad = jnp.pad(emb_table, ((0, zcap - emb_table.shape[0]), (0, 0)))
    W1 = W_cat[:emb]
    W2 = W_cat[emb:2 * emb]
    W3 = W_cat[2 * emb:]
    freq2 = freq.reshape(1, r)
    s2 = jnp.arange(s, dtype=jnp.float32).reshape(1, s)
    Wr = W_sbf1.reshape(s, r, nb).transpose(1, 0, 2).reshape(r, s * nb)
    Wrp = jnp.pad(Wr, ((0, 0), (0, emb - s * nb)))
    Wbil2 = W_bil.transpose(0, 2, 1).reshape(emb, nb * emb)

    z2 = z.astype(jnp.int32).reshape(n, 1)
    d2 = d.reshape(e, 1)
    ang2 = angle.reshape(t, 1)

    bn = _pick(n)
    be = _pick(e)
    bt = _pick(t)

    x = _node_embed(z2, embpad, bn)

    both = _gather_rows(x, jnp.concatenate([src, dst]).astype(jnp.int32))
    xs = both[:e]
    xd = both[e:]

    m, m_ji, mkx, rbo = _edge_block(d2, xs, xd, freq2, W_rbf_emb, W1, W2, W3,
                                    W_ji, W_kj, W_rbf1, W_rbf_out, Wrp, be)

    xkx = _gather_rows(mkx, idx_kj.astype(jnp.int32))

    tr = _triplet_block(xkx, ang2, s2, Wbil2, bt)

    agg = _scatter_add_rows(tr, idx_ji.astype(jnp.int32), e)

    g = _post_block(m, m_ji, agg, rbo, W_res1a, W_res1b, W_res2a, W_res2b, be)

    node = _scatter_add_rows(g, dst.astype(jnp.int32), n)

    return _out_block(node, W_out1, W_out2, bn)
